# baseline (speedup 1.0000x reference)
import functools

import jax
import jax.numpy as jnp
from jax.experimental import pallas as pl

ROWS, COLS = 128, 100000
ROW_BLOCK = 8


def _threefry_bits_xor(flat_u32):
    k0 = jnp.uint32(0)
    k1 = jnp.uint32(1)
    ks2 = jnp.uint32(0x1BD11BDA) ^ k0 ^ k1
    rot1 = (13, 15, 26, 6)
    rot2 = (17, 29, 16, 24)

    def rotl(x, r):
        return (x << jnp.uint32(r)) | (x >> jnp.uint32(32 - r))

    x0 = jnp.zeros_like(flat_u32) + k0
    x1 = flat_u32 + k1
    inject = ((k1, ks2, 1), (ks2, k0, 2), (k0, k1, 3), (k1, ks2, 4),
              (ks2, k0, 5))
    for i in range(5):
        for r in (rot1 if i % 2 == 0 else rot2):
            x0 = x0 + x1
            x1 = rotl(x1, r)
            x1 = x1 ^ x0
        a, b, c = inject[i]
        x0 = x0 + a
        x1 = x1 + b + jnp.uint32(c)
    return x0 ^ x1


def _gumbel_softmax_block(logits_ref, out_ref):
    i = pl.program_id(0)
    shape = logits_ref.shape

    row = jax.lax.broadcasted_iota(jnp.uint32, shape, 0)
    col = jax.lax.broadcasted_iota(jnp.uint32, shape, 1)
    base = (jnp.uint32(i) * jnp.uint32(ROW_BLOCK)) * jnp.uint32(COLS)
    flat = base + row * jnp.uint32(COLS) + col

    bits = _threefry_bits_xor(flat)

    fbits = jax.lax.bitcast_convert_type(
        (bits >> jnp.uint32(9)) | jnp.uint32(0x3F800000), jnp.float32)
    minval = jnp.float32(1e-10)
    u = fbits - jnp.float32(1.0)
    u = u * (jnp.float32(1.0) - minval) + minval
    u = jnp.maximum(u, minval)

    noise = -jnp.log(-jnp.log(u))
    z = logits_ref[...] + noise

    m = jnp.max(z, axis=-1, keepdims=True)
    e = jnp.exp(z - m)
    s = jnp.sum(e, axis=-1, keepdims=True)
    out_ref[...] = e / s


@jax.jit
def kernel(logits):
    grid = (ROWS // ROW_BLOCK,)
    return pl.pallas_call(
        _gumbel_softmax_block,
        grid=grid,
        in_specs=[pl.BlockSpec((ROW_BLOCK, COLS), lambda i: (i, 0))],
        out_specs=pl.BlockSpec((ROW_BLOCK, COLS), lambda i: (i, 0)),
        out_shape=jax.ShapeDtypeStruct((ROWS, COLS), jnp.float32),
    )(logits)
